# baseline (speedup 1.0000x reference)
import functools

import jax
import jax.numpy as jnp
from jax import lax
from jax.experimental import pallas as pl
from jax.experimental.pallas import tpu as pltpu
from jax.experimental.pallas import tpu_sc as plsc

_IDXW = 128


@functools.lru_cache(maxsize=None)
def _make_gather(V, D, B):
    info = plsc.get_sparse_core_info()
    NC, NS = info.num_cores, info.num_subcores
    NW = NC * NS
    assert B % (NW * _IDXW) == 0
    rows_per_w = B // NW
    K = 4
    CH = K * _IDXW
    n_chunks = rows_per_w // CH
    idx_rows_per_w = rows_per_w // _IDXW

    mesh = plsc.VectorSubcoreMesh(core_axis_name="c", subcore_axis_name="s")

    @functools.partial(
        pl.kernel,
        mesh=mesh,
        compiler_params=pltpu.CompilerParams(use_tc_tiling_on_sc=False),
        out_type=jax.ShapeDtypeStruct((B, D), jnp.float32),
        scratch_types=[
            pltpu.VMEM((K, _IDXW), jnp.int32),
            pltpu.VMEM((CH, D), jnp.float32),
            pltpu.SemaphoreType.DMA,
        ],
    )
    def gather_kernel(table_hbm, idx_hbm, out_hbm, idx_v, rows_v, sem):
        wid = lax.axis_index("s") * NC + lax.axis_index("c")
        row0 = wid * rows_per_w
        irow0 = wid * idx_rows_per_w

        def chunk(c, carry):
            pltpu.sync_copy(idx_hbm.at[pl.ds(irow0 + c * K, K)], idx_v)
            copies = [
                pltpu.async_copy(
                    table_hbm.at[idx_v.at[j]],
                    rows_v.at[pl.ds(j * _IDXW, _IDXW)],
                    sem,
                )
                for j in range(K)
            ]
            for cp in copies:
                cp.wait()
            pltpu.sync_copy(rows_v, out_hbm.at[pl.ds(row0 + c * CH, CH)])
            return carry

        lax.fori_loop(0, n_chunks, chunk, 0)

    return gather_kernel


def kernel(src_seq, src_mask, emb_table):
    Bt, S = src_seq.shape
    V, D = emb_table.shape
    B = Bt * S
    idx2d = src_seq.reshape(B // _IDXW, _IDXW)
    out = _make_gather(V, D, B)(emb_table, idx2d)
    return out.reshape(Bt, S, D)


# speedup vs baseline: 1.0454x; 1.0454x over previous
import functools

import jax
import jax.numpy as jnp
from jax import lax
from jax.experimental import pallas as pl
from jax.experimental.pallas import tpu as pltpu
from jax.experimental.pallas import tpu_sc as plsc

_IDXW = 128


@functools.lru_cache(maxsize=None)
def _make_gather(V, D, B):
    info = plsc.get_sparse_core_info()
    NC, NS = info.num_cores, info.num_subcores
    NW = NC * NS
    assert B % (NW * _IDXW) == 0
    rows_per_w = B // NW
    K = 2
    R = 4
    CH = K * _IDXW
    n_chunks = rows_per_w // CH
    n_super = n_chunks // R
    assert n_chunks % R == 0 and n_super >= 2
    idx_rows_per_w = rows_per_w // _IDXW

    mesh = plsc.VectorSubcoreMesh(core_axis_name="c", subcore_axis_name="s")

    @functools.partial(
        pl.kernel,
        mesh=mesh,
        compiler_params=pltpu.CompilerParams(use_tc_tiling_on_sc=False),
        out_type=jax.ShapeDtypeStruct((B, D), jnp.float32),
        scratch_types=[
            pltpu.VMEM((R, K, _IDXW), jnp.int32),
            pltpu.VMEM((R, CH, D), jnp.float32),
            pltpu.SemaphoreType.DMA((R,)),
            pltpu.SemaphoreType.DMA((R,)),
            pltpu.SemaphoreType.DMA((R,)),
        ],
    )
    def gather_kernel(table_hbm, idx_hbm, out_hbm, idx_v, rows_v,
                      idx_sem, gat_sem, out_sem):
        wid = lax.axis_index("s") * NC + lax.axis_index("c")
        row0 = wid * rows_per_w
        irow0 = wid * idx_rows_per_w

        def start_idx(c, u):
            pltpu.async_copy(
                idx_hbm.at[pl.ds(irow0 + c * K, K)], idx_v.at[u],
                idx_sem.at[u])

        def wait_idx(u):
            pltpu.make_async_copy(
                idx_hbm.at[pl.ds(irow0, K)], idx_v.at[u],
                idx_sem.at[u]).wait()

        def fire(c, u):
            for j in range(K):
                pltpu.async_copy(
                    table_hbm.at[idx_v.at[u, j]],
                    rows_v.at[u, pl.ds(j * _IDXW, _IDXW)],
                    gat_sem.at[u])

        def wait_gat(u):
            pltpu.make_async_copy(
                out_hbm.at[pl.ds(row0, CH)], rows_v.at[u],
                gat_sem.at[u]).wait()

        def start_out(c, u):
            pltpu.async_copy(
                rows_v.at[u], out_hbm.at[pl.ds(row0 + c * CH, CH)],
                out_sem.at[u])

        def wait_out(u):
            pltpu.make_async_copy(
                rows_v.at[u], out_hbm.at[pl.ds(row0, CH)],
                out_sem.at[u]).wait()

        for u in range(R):
            start_idx(u, u)
        for u in range(R):
            wait_idx(u)
            fire(u, u)
        for u in range(R):
            wait_gat(u)
            start_out(u, u)
            start_idx(R + u, u)

        def super_body(s, carry):
            c0 = s * R
            for u in range(R):
                wait_idx(u)
                wait_out(u)
                fire(c0 + u, u)
            for u in range(R):
                wait_gat(u)
                start_out(c0 + u, u)
                start_idx(c0 + R + u, u)
            return carry

        lax.fori_loop(1, n_super - 1, super_body, 0)

        cl = (n_super - 1) * R
        for u in range(R):
            wait_idx(u)
            wait_out(u)
            fire(cl + u, u)
        for u in range(R):
            wait_gat(u)
            start_out(cl + u, u)
        for u in range(R):
            wait_out(u)

    return gather_kernel


def kernel(src_seq, src_mask, emb_table):
    Bt, S = src_seq.shape
    V, D = emb_table.shape
    B = Bt * S
    idx2d = src_seq.reshape(B // _IDXW, _IDXW)
    out = _make_gather(V, D, B)(emb_table, idx2d)
    return out.reshape(Bt, S, D)


# speedup vs baseline: 1.0482x; 1.0027x over previous
import functools

import jax
import jax.numpy as jnp
from jax import lax
from jax.experimental import pallas as pl
from jax.experimental.pallas import tpu as pltpu
from jax.experimental.pallas import tpu_sc as plsc


@functools.lru_cache(maxsize=None)
def _make_gather(V, D, Bt, S):
    info = plsc.get_sparse_core_info()
    NC, NS = info.num_cores, info.num_subcores
    NW = NC * NS
    assert Bt % NW == 0
    b_per_w = Bt // NW
    R = 4
    n_super = b_per_w // R
    assert b_per_w % R == 0 and n_super >= 2
    splits = []
    off = 0
    while off < S:
        n = min(128, S - off)
        splits.append((off, n))
        off += n

    mesh = plsc.VectorSubcoreMesh(core_axis_name="c", subcore_axis_name="s")

    @functools.partial(
        pl.kernel,
        mesh=mesh,
        compiler_params=pltpu.CompilerParams(use_tc_tiling_on_sc=False),
        out_type=jax.ShapeDtypeStruct((Bt, S, D), jnp.float32),
        scratch_types=[
            pltpu.VMEM((R, S), jnp.int32),
            pltpu.VMEM((R, S, D), jnp.float32),
            pltpu.SemaphoreType.DMA((R,)),
            pltpu.SemaphoreType.DMA((R,)),
            pltpu.SemaphoreType.DMA((R,)),
        ],
    )
    def gather_kernel(table_hbm, idx_hbm, out_hbm, idx_v, rows_v,
                      idx_sem, gat_sem, out_sem):
        wid = lax.axis_index("s") * NC + lax.axis_index("c")
        b0 = wid * b_per_w

        def start_idx(b, u):
            pltpu.async_copy(idx_hbm.at[b0 + b], idx_v.at[u], idx_sem.at[u])

        def wait_idx(u):
            pltpu.make_async_copy(
                idx_hbm.at[b0], idx_v.at[u], idx_sem.at[u]).wait()

        def fire(b, u):
            for (o, n) in splits:
                pltpu.async_copy(
                    table_hbm.at[idx_v.at[u, pl.ds(o, n)]],
                    rows_v.at[u, pl.ds(o, n)],
                    gat_sem.at[u])

        def wait_gat(u):
            pltpu.make_async_copy(
                out_hbm.at[b0], rows_v.at[u], gat_sem.at[u]).wait()

        def start_out(b, u):
            pltpu.async_copy(rows_v.at[u], out_hbm.at[b0 + b], out_sem.at[u])

        def wait_out(u):
            pltpu.make_async_copy(
                rows_v.at[u], out_hbm.at[b0], out_sem.at[u]).wait()

        for u in range(R):
            start_idx(u, u)
        for u in range(R):
            wait_idx(u)
            fire(u, u)
        for u in range(R):
            wait_gat(u)
            start_out(u, u)
            start_idx(R + u, u)

        def super_body(s, carry):
            c0 = s * R
            for u in range(R):
                wait_idx(u)
                wait_out(u)
                fire(c0 + u, u)
            for u in range(R):
                wait_gat(u)
                start_out(c0 + u, u)
                start_idx(c0 + R + u, u)
            return carry

        lax.fori_loop(1, n_super - 1, super_body, 0)

        cl = (n_super - 1) * R
        for u in range(R):
            wait_idx(u)
            wait_out(u)
            fire(cl + u, u)
        for u in range(R):
            wait_gat(u)
            start_out(cl + u, u)
        for u in range(R):
            wait_out(u)

    return gather_kernel


def kernel(src_seq, src_mask, emb_table):
    Bt, S = src_seq.shape
    V, D = emb_table.shape
    return _make_gather(V, D, Bt, S)(emb_table, src_seq)
